# baseline (speedup 1.0000x reference)
import functools

import jax
import jax.numpy as jnp
from jax import lax
from jax.experimental import pallas as pl
from jax.experimental.pallas import tpu as pltpu
from jax.experimental.pallas import tpu_sc as plsc

NC = 2
NS = 16
NW = NC * NS
L = 16


def _make_gather(B, H, V, D):
    bw = B // NW
    ng = bw // L
    mesh = plsc.VectorSubcoreMesh(core_axis_name="c", subcore_axis_name="s")

    @functools.partial(
        pl.kernel,
        mesh=mesh,
        out_type=jax.ShapeDtypeStruct((H, D, B), jnp.float32),
        scratch_types=[
            pltpu.VMEM((H, bw), jnp.int32),
            pltpu.VMEM((2, bw, D), jnp.float32),
            pltpu.VMEM((2, D * bw), jnp.float32),
            pltpu.SemaphoreType.DMA,
            pltpu.SemaphoreType.DMA,
        ],
        compiler_params=pltpu.CompilerParams(
            use_tc_tiling_on_sc=False, needs_layout_passes=False
        ),
    )
    def k(idx_hbm, table_hbm, out_hbm, idx_v, buf, bt, sem_g, sem_s):
        wid = lax.axis_index("s") * NC + lax.axis_index("c")
        col0 = wid * bw
        pltpu.sync_copy(idx_hbm.at[:, pl.ds(col0, bw)], idx_v)

        lanes = lax.iota(jnp.int32, L)
        idx_lo = lanes * bw
        idx_hi = idx_lo + L * bw

        def fire(h, p):
            pltpu.async_copy(table_hbm.at[idx_v.at[h]], buf.at[p], sem_g)

        def wait_gather(p):
            pltpu.make_async_copy(
                table_hbm.at[pl.ds(0, bw)], buf.at[p], sem_g
            ).wait()

        def transpose(p):
            def body(c, carry):
                v_lo = buf[p, c, pl.ds(0, L)]
                v_hi = buf[p, c, pl.ds(L, L)]
                plsc.store_scatter(bt.at[p], [idx_lo + c], v_lo)
                plsc.store_scatter(bt.at[p], [idx_hi + c], v_hi)
                return carry

            lax.fori_loop(0, bw, body, 0, unroll=8)

        def store(h, p):
            for d in range(D):
                pltpu.async_copy(
                    bt.at[p, pl.ds(d * bw, bw)],
                    out_hbm.at[h, d, pl.ds(col0, bw)],
                    sem_s,
                )

        def wait_store(p):
            for d in range(D):
                pltpu.make_async_copy(
                    bt.at[p, pl.ds(d * bw, bw)],
                    out_hbm.at[0, 0, pl.ds(col0, bw)],
                    sem_s,
                ).wait()

        fire(0, 0)
        fire(1, 1)

        def body(i, carry):
            h0 = i * 2

            wait_gather(0)

            @pl.when(i > 0)
            def _():
                wait_store(0)

            transpose(0)

            @pl.when(i < H // 2 - 1)
            def _():
                fire(h0 + 2, 0)

            store(h0, 0)

            wait_gather(1)

            @pl.when(i > 0)
            def _():
                wait_store(1)

            transpose(1)

            @pl.when(i < H // 2 - 1)
            def _():
                fire(h0 + 3, 1)

            store(h0 + 1, 1)
            return carry

        lax.fori_loop(0, H // 2, body, 0, unroll=False)
        wait_store(0)
        wait_store(1)

    return k


def kernel(x, W):
    B, H = x.shape
    V, D = W.shape
    out_t = _make_gather(B, H, V, D)(x.T.astype(jnp.int32), W)
    return jnp.transpose(out_t, (2, 0, 1))
